# baseline (speedup 1.0000x reference)
import functools

import jax
import jax.numpy as jnp
from jax import lax
from jax.experimental import pallas as pl
from jax.experimental.pallas import tpu as pltpu
from jax.experimental.pallas import tpu_sc as plsc

USER_LEN = 1000000
MOVIE_LEN = 100000
EMBED = 32
BATCH = 16384

RRELU_SLOPE = (1.0 / 8.0 + 1.0 / 3.0) / 2.0

_INFO = plsc.get_sparse_core_info()
_NC = _INFO.num_cores
_NS = _INFO.num_subcores
_NW = _NC * _NS
_B_PER_W = BATCH // _NW
_G = 16
_NGROUP = _B_PER_W // _G
_CHUNK = 128
_NCHUNK = _B_PER_W // _CHUNK


def _rrelu(x):
    return jnp.where(x >= 0, x, x * RRELU_SLOPE)


def _sc_kernel(seq0_hbm, seq1_hbm, w0t_hbm, w1t_hbm, b0_hbm, b1_hbm,
               out_hbm,
               idx0_v, idx1_v, blk0_v, blk1_v,
               cols0_v, cols1_v, bias0_v, bias1_v, out_v,
               sem0, sem1, semb):
    wid = lax.axis_index("s") * _NC + lax.axis_index("c")
    base = wid * _B_PER_W

    pltpu.sync_copy(seq0_hbm.at[pl.ds(base, _B_PER_W)], idx0_v)
    pltpu.sync_copy(seq1_hbm.at[pl.ds(base, _B_PER_W)], idx1_v)

    bias_copies = []
    for j in range(_NCHUNK):
        sl = pl.ds(j * _CHUNK, _CHUNK)
        bias_copies.append(pltpu.async_copy(
            b0_hbm.at[idx0_v.at[sl]], bias0_v.at[sl], semb))
        bias_copies.append(pltpu.async_copy(
            b1_hbm.at[idx1_v.at[sl]], bias1_v.at[sl], semb))

    lane = lax.iota(jnp.int32, 16)

    def blk_copy(table_hbm, blk_v, sem, s, slot):
        sa = pl.multiple_of((s >> 3) << 3, 8)
        return pltpu.make_async_copy(
            table_hbm.at[:, pl.ds(sa, 8)], blk_v.at[slot], sem)

    def enqueue_group(g, parity):
        v0 = idx0_v[pl.ds(g * _G, _G)]
        v1 = idx1_v[pl.ds(g * _G, _G)]
        for j in range(_G):
            slot = parity * _G + j
            blk_copy(w0t_hbm, blk0_v, sem0, v0[j], slot).start()
            blk_copy(w1t_hbm, blk1_v, sem1, v1[j], slot).start()

    def wait_extract_group(g, parity):
        v0 = idx0_v[pl.ds(g * _G, _G)]
        v1 = idx1_v[pl.ds(g * _G, _G)]
        for j in range(_G):
            slot = parity * _G + j
            blk_copy(w0t_hbm, blk0_v, sem0, v0[j], slot).wait()
            blk_copy(w1t_hbm, blk1_v, sem1, v1[j], slot).wait()
            k = g * _G + j
            for (blk_v, cols_v, s) in ((blk0_v, cols0_v, v0[j]),
                                       (blk1_v, cols1_v, v1[j])):
                c = jnp.full((16,), s & 7, jnp.int32)
                sv = jnp.full((16,), slot, jnp.int32)
                cols_v[pl.ds(k * EMBED, 16)] = plsc.load_gather(
                    blk_v, [sv, lane, c])
                cols_v[pl.ds(k * EMBED + 16, 16)] = plsc.load_gather(
                    blk_v, [sv, lane + 16, c])

    enqueue_group(0, 0)

    def pipe(g, carry):
        par = lax.rem(g, 2)
        enqueue_group(g, par)
        wait_extract_group(g - 1, 1 - par)
        return carry

    lax.fori_loop(1, _NGROUP, pipe, 0, unroll=False)
    wait_extract_group(_NGROUP - 1, lax.rem(_NGROUP - 1, 2))

    for c in bias_copies:
        c.wait()

    def body(g, carry):
        b16 = g * 16
        flat = (b16 + lane) * EMBED
        acc = bias0_v[pl.ds(b16, 16)] + bias1_v[pl.ds(b16, 16)]
        for e in range(EMBED):
            g0 = plsc.load_gather(cols0_v, [flat + e])
            g1 = plsc.load_gather(cols1_v, [flat + e])
            acc = acc + _rrelu(g0) * _rrelu(g1)
        out_v[pl.ds(b16, 16)] = acc
        return carry

    lax.fori_loop(0, _NGROUP, body, 0)

    pltpu.sync_copy(out_v, out_hbm.at[pl.ds(base, _B_PER_W)])


@functools.partial(
    pl.kernel,
    out_type=jax.ShapeDtypeStruct((BATCH,), jnp.float32),
    mesh=plsc.VectorSubcoreMesh(core_axis_name="c", subcore_axis_name="s"),
    compiler_params=pltpu.CompilerParams(
        needs_layout_passes=False, use_tc_tiling_on_sc=False),
    scratch_types=[
        pltpu.VMEM((_B_PER_W,), jnp.int32),
        pltpu.VMEM((_B_PER_W,), jnp.int32),
        pltpu.VMEM((2 * _G, EMBED, 8), jnp.float32),
        pltpu.VMEM((2 * _G, EMBED, 8), jnp.float32),
        pltpu.VMEM((_B_PER_W * EMBED,), jnp.float32),
        pltpu.VMEM((_B_PER_W * EMBED,), jnp.float32),
        pltpu.VMEM((_B_PER_W,), jnp.float32),
        pltpu.VMEM((_B_PER_W,), jnp.float32),
        pltpu.VMEM((_B_PER_W,), jnp.float32),
        pltpu.SemaphoreType.DMA,
        pltpu.SemaphoreType.DMA,
        pltpu.SemaphoreType.DMA,
    ],
)
def _mf_forward(seq0, seq1, w0t, w1t, b0, b1, out, *scratch):
    _sc_kernel(seq0, seq1, w0t, w1t, b0, b1, out, *scratch)


def kernel(seq0, seq1, W0, W1, B0, B1):
    seq0 = seq0.astype(jnp.int32)
    seq1 = seq1.astype(jnp.int32)
    out = _mf_forward(seq0, seq1, W0.T, W1.T, B0.reshape(-1), B1.reshape(-1))
    return out.reshape(BATCH, 1)


# speedup vs baseline: 3.1360x; 3.1360x over previous
import functools

import jax
import jax.numpy as jnp
from jax import lax
from jax.experimental import pallas as pl
from jax.experimental.pallas import tpu as pltpu
from jax.experimental.pallas import tpu_sc as plsc

USER_LEN = 1000000
MOVIE_LEN = 100000
EMBED = 32
BATCH = 16384

RRELU_SLOPE = (1.0 / 8.0 + 1.0 / 3.0) / 2.0

_INFO = plsc.get_sparse_core_info()
_NC = _INFO.num_cores
_NS = _INFO.num_subcores
_NW = _NC * _NS
_B_PER_W = BATCH // _NW
_CHUNK = 128
_NCHUNK = _B_PER_W // _CHUNK
_TC = 128


def _rrelu(x):
    return jnp.where(x >= 0, x, x * RRELU_SLOPE)


def _make_detile(n_rows):
    n_cols = (n_rows + _TC - 1) // _TC
    n_pad = n_cols * _TC
    per_w = (n_cols + _NW - 1) // _NW

    def body(wt_hbm, flat_hbm, blk_v, out_v, sem_in, sem_out):
        wid = lax.axis_index("s") * _NC + lax.axis_index("c")
        lo = wid * per_w
        hi = jnp.minimum(lo + per_w, n_cols)
        lane = lax.iota(jnp.int32, 16)

        def copy_in(tc, slot):
            return pltpu.make_async_copy(
                wt_hbm.at[:, pl.ds(pl.multiple_of(tc * _TC, _TC), _TC)],
                blk_v.at[slot], sem_in)

        def copy_out(tc, slot):
            return pltpu.make_async_copy(
                out_v.at[pl.ds(slot * (_TC * EMBED), _TC * EMBED)],
                flat_hbm.at[pl.ds(tc * _TC * EMBED, _TC * EMBED)], sem_out)

        @pl.when(lo < hi)
        def _():
            copy_in(lo, 0).start()

            def step(tc, carry):
                par = lax.rem(tc - lo, 2)

                @pl.when(tc + 1 < hi)
                def _():
                    copy_in(tc + 1, 1 - par).start()

                copy_in(tc, par).wait()

                @pl.when(tc - 2 >= lo)
                def _():
                    copy_out(tc - 2, par).wait()

                sv = jnp.full((16,), par, jnp.int32)

                def trans(c, carry2):
                    cc = jnp.full((16,), c, jnp.int32)
                    o = c * EMBED + par * (_TC * EMBED)
                    out_v[pl.ds(o, 16)] = plsc.load_gather(
                        blk_v, [sv, lane, cc])
                    out_v[pl.ds(o + 16, 16)] = plsc.load_gather(
                        blk_v, [sv, lane + 16, cc])
                    return carry2

                lax.fori_loop(0, _TC, trans, 0)
                copy_out(tc, par).start()
                return carry

            lax.fori_loop(lo, hi, step, 0)

            @pl.when(hi - 2 >= lo)
            def _():
                copy_out(hi - 2, lax.rem(hi - 2 - lo, 2)).wait()

            copy_out(hi - 1, lax.rem(hi - 1 - lo, 2)).wait()

    return functools.partial(
        pl.kernel,
        out_type=jax.ShapeDtypeStruct((n_pad * EMBED,), jnp.float32),
        mesh=plsc.VectorSubcoreMesh(
            core_axis_name="c", subcore_axis_name="s"),
        compiler_params=pltpu.CompilerParams(needs_layout_passes=False),
        scratch_types=[
            pltpu.VMEM((2, EMBED, _TC), jnp.float32),
            pltpu.VMEM((2 * _TC * EMBED,), jnp.float32),
            pltpu.SemaphoreType.DMA,
            pltpu.SemaphoreType.DMA,
        ],
    )(lambda wt, flat, *s: body(wt, flat, *s)), n_pad


_detile_w0, _W0_PAD = _make_detile(USER_LEN + 1)
_detile_w1, _W1_PAD = _make_detile(MOVIE_LEN + 1)


def _sc_lookup(seq0_hbm, seq1_hbm, w0_hbm, w1_hbm, b0_hbm, b1_hbm,
               out_hbm,
               idx0_v, idx1_v, rows0_v, rows1_v, bias0_v, bias1_v,
               out_v, sem):
    wid = lax.axis_index("s") * _NC + lax.axis_index("c")

    row0 = wid * _NCHUNK
    pltpu.sync_copy(seq0_hbm.at[pl.ds(row0, _NCHUNK)], idx0_v)
    pltpu.sync_copy(seq1_hbm.at[pl.ds(row0, _NCHUNK)], idx1_v)

    copies = []
    for j in range(_NCHUNK):
        dst = pl.ds(j * _CHUNK, _CHUNK)
        copies.append(pltpu.async_copy(
            w0_hbm.at[idx0_v.at[j]], rows0_v.at[dst], sem))
        copies.append(pltpu.async_copy(
            w1_hbm.at[idx1_v.at[j]], rows1_v.at[dst], sem))
        copies.append(pltpu.async_copy(
            b0_hbm.at[idx0_v.at[j]], bias0_v.at[dst], sem))
        copies.append(pltpu.async_copy(
            b1_hbm.at[idx1_v.at[j]], bias1_v.at[dst], sem))
    for c in copies:
        c.wait()

    lane = lax.iota(jnp.int32, 16)

    def body(g, carry):
        base = g * 16
        item = base + lane
        acc = bias0_v[pl.ds(base, 16)] + bias1_v[pl.ds(base, 16)]
        for e in range(EMBED):
            ee = jnp.full((16,), e, jnp.int32)
            g0 = plsc.load_gather(rows0_v, [item, ee])
            g1 = plsc.load_gather(rows1_v, [item, ee])
            acc = acc + _rrelu(g0) * _rrelu(g1)
        out_v[pl.ds(base, 16)] = acc
        return carry

    lax.fori_loop(0, _B_PER_W // 16, body, 0)

    pltpu.sync_copy(out_v, out_hbm.at[pl.ds(wid * _B_PER_W, _B_PER_W)])


@functools.partial(
    pl.kernel,
    out_type=jax.ShapeDtypeStruct((BATCH,), jnp.float32),
    mesh=plsc.VectorSubcoreMesh(core_axis_name="c", subcore_axis_name="s"),
    compiler_params=pltpu.CompilerParams(
        needs_layout_passes=False, use_tc_tiling_on_sc=False),
    scratch_types=[
        pltpu.VMEM((_NCHUNK, _CHUNK), jnp.int32),
        pltpu.VMEM((_NCHUNK, _CHUNK), jnp.int32),
        pltpu.VMEM((_B_PER_W, EMBED), jnp.float32),
        pltpu.VMEM((_B_PER_W, EMBED), jnp.float32),
        pltpu.VMEM((_B_PER_W,), jnp.float32),
        pltpu.VMEM((_B_PER_W,), jnp.float32),
        pltpu.VMEM((_B_PER_W,), jnp.float32),
        pltpu.SemaphoreType.DMA,
    ],
)
def _mf_forward(seq0, seq1, w0, w1, b0, b1, out, *scratch):
    _sc_lookup(seq0, seq1, w0, w1, b0, b1, out, *scratch)


def kernel(seq0, seq1, W0, W1, B0, B1):
    seq0 = seq0.astype(jnp.int32).reshape(BATCH // _CHUNK, _CHUNK)
    seq1 = seq1.astype(jnp.int32).reshape(BATCH // _CHUNK, _CHUNK)
    w0_flat = _detile_w0(W0.T)
    w1_flat = _detile_w1(W1.T)
    w0 = w0_flat.reshape(_W0_PAD, EMBED)
    w1 = w1_flat.reshape(_W1_PAD, EMBED)
    out = _mf_forward(seq0, seq1, w0, w1, B0.reshape(-1), B1.reshape(-1))
    return out.reshape(BATCH, 1)
